# baseline (speedup 1.0000x reference)
import functools

import jax
import jax.numpy as jnp
from jax import lax
from jax.experimental import pallas as pl
from jax.experimental.pallas import tpu as pltpu
from jax.experimental.pallas import tpu_sc as plsc

_NC = 2
_NS = 16
_NW = _NC * _NS
_L = 16

_B = 16384
_T = 200
_Q = 4
_BQ = _B // _Q
_TASKS = _T * _Q
_PER_W = _TASKS // _NW
_TRIPLES = (_PER_W - 1) // 3
_GROUPS = _BQ // _L


def _task_coords(tid):
    t = tid // _Q
    b0 = (tid % _Q) * _BQ
    return t, b0


def _body(g_hbm, tbl_hbm, out_hbm,
          gv0, gv1, gv2, ov0, ov1, ov2, tv,
          isem0, isem1, isem2, osem0, osem1, osem2):
    wid = lax.axis_index("s") * _NC + lax.axis_index("c")
    task0 = wid * _PER_W
    gvs = (gv0, gv1, gv2)
    ovs = (ov0, ov1, ov2)
    isems = (isem0, isem1, isem2)
    osems = (osem0, osem1, osem2)

    def in_copy(tid, buf):
        t, b0 = _task_coords(tid)
        return pltpu.make_async_copy(g_hbm.at[t, pl.ds(b0, _BQ)], gvs[buf],
                                     isems[buf])

    def out_copy(tid, buf):
        t, b0 = _task_coords(tid)
        return pltpu.make_async_copy(ovs[buf], out_hbm.at[t, :, pl.ds(b0, _BQ)],
                                     osems[buf])

    in_copy(task0, 0).start()
    pltpu.sync_copy(tbl_hbm, tv)
    cvecs = tuple(jnp.full((_L,), c, jnp.int32) for c in range(4))

    def compute(buf):
        gv = gvs[buf]
        ov = ovs[buf]

        @plsc.parallel_loop(0, _GROUPS, unroll=8)
        def grp(i):
            gvec = gv[pl.ds(i * _L, _L)]
            for c in range(4):
                ov[c, pl.ds(i * _L, _L)] = plsc.load_gather(
                    tv, [gvec, cvecs[c]])

    in_copy(task0, 0).wait()
    in_copy(task0 + 1, 1).start()
    compute(0)
    out_copy(task0, 0).start()

    def triple(j, carry):
        for p in range(3):
            tid = task0 + 1 + 3 * j + p
            buf = (1 + p) % 3
            in_copy(tid, buf).wait()
            nxt = (buf + 1) % 3
            if p < 2:
                in_copy(tid + 1, nxt).start()
            else:
                @pl.when(j < _TRIPLES - 1)
                def _():
                    in_copy(tid + 1, nxt).start()
            if p == 2:
                out_copy(tid, buf).wait()
            else:
                @pl.when(j > 0)
                def _():
                    out_copy(tid, buf).wait()
            compute(buf)
            out_copy(tid, buf).start()
        return carry

    lax.fori_loop(0, _TRIPLES, triple, 0)
    for tid in (_PER_W - 3, _PER_W - 2, _PER_W - 1):
        out_copy(task0 + tid, tid % 3).wait()


@jax.jit
def kernel(g, table):
    mesh = plsc.VectorSubcoreMesh(core_axis_name="c", subcore_axis_name="s")
    run = pl.kernel(
        _body,
        mesh=mesh,
        out_type=jax.ShapeDtypeStruct((_T, 4, _B), jnp.float32),
        scratch_types=[
            pltpu.VMEM((_BQ,), jnp.int32),
            pltpu.VMEM((_BQ,), jnp.int32),
            pltpu.VMEM((_BQ,), jnp.int32),
            pltpu.VMEM((4, _BQ), jnp.float32),
            pltpu.VMEM((4, _BQ), jnp.float32),
            pltpu.VMEM((4, _BQ), jnp.float32),
            pltpu.VMEM((5, 4), jnp.float32),
            pltpu.SemaphoreType.DMA,
            pltpu.SemaphoreType.DMA,
            pltpu.SemaphoreType.DMA,
            pltpu.SemaphoreType.DMA,
            pltpu.SemaphoreType.DMA,
            pltpu.SemaphoreType.DMA,
        ],
        compiler_params=pltpu.CompilerParams(needs_layout_passes=False),
    )
    outP = run(g.T, table)
    return outP.transpose(2, 0, 1)


# speedup vs baseline: 4.6582x; 4.6582x over previous
import functools

import jax
import jax.numpy as jnp
from jax import lax
from jax.experimental import pallas as pl
from jax.experimental.pallas import tpu as pltpu
from jax.experimental.pallas import tpu_sc as plsc

_NC = 2
_NS = 16
_NW = _NC * _NS
_L = 16

_B = 16384
_T = 200
_Q = 4
_BQ = _B // _Q
_TASKS = _T * _Q
_PER_W = _TASKS // _NW
_PAIRS = (_PER_W - 1) // 2
_GROUPS = _BQ // _L
_CSTRIDE = 16


def _task_coords(tid):
    t = tid // _Q
    b0 = (tid % _Q) * _BQ
    return t, b0


def _body(g_hbm, tbl_hbm, out_hbm,
          gv0, gv1, ov0, ov1, tv0,
          isem0, isem1, osem0, osem1):
    wid = lax.axis_index("s") * _NC + lax.axis_index("c")
    task0 = wid * _PER_W

    def in_copy(tid, gv, isem):
        t, b0 = _task_coords(tid)
        return pltpu.make_async_copy(g_hbm.at[t, pl.ds(b0, _BQ)], gv, isem)

    def out_copy(tid, ov, osem):
        t, b0 = _task_coords(tid)
        return pltpu.make_async_copy(ov, out_hbm.at[t, :, pl.ds(b0, _BQ)],
                                     osem)

    in_copy(task0, gv0, isem0).start()
    pltpu.sync_copy(tbl_hbm, tv0)

    def compute(gv, ov):
        @plsc.parallel_loop(0, _GROUPS, unroll=8)
        def grp(i):
            gvec = gv[pl.ds(i * _L, _L)]
            for c in range(4):
                idx = gvec if c == 0 else gvec + (c * _CSTRIDE)
                ov[c, pl.ds(i * _L, _L)] = plsc.load_gather(tv0, [idx])

    in_copy(task0, gv0, isem0).wait()
    in_copy(task0 + 1, gv1, isem1).start()
    compute(gv0, ov0)
    out_copy(task0, ov0, osem0).start()

    def pair(j, carry):
        t1 = task0 + 1 + 2 * j
        in_copy(t1, gv1, isem1).wait()
        in_copy(t1 + 1, gv0, isem0).start()

        @pl.when(j > 0)
        def _():
            out_copy(t1, ov1, osem1).wait()

        compute(gv1, ov1)
        out_copy(t1, ov1, osem1).start()

        t2 = t1 + 1
        in_copy(t2, gv0, isem0).wait()

        @pl.when(j < _PAIRS - 1)
        def _():
            in_copy(t2 + 1, gv1, isem1).start()

        out_copy(t2, ov0, osem0).wait()
        compute(gv0, ov0)
        out_copy(t2, ov0, osem0).start()
        return carry

    lax.fori_loop(0, _PAIRS, pair, 0)
    out_copy(task0 + _PER_W - 2, ov1, osem1).wait()
    out_copy(task0 + _PER_W - 1, ov0, osem0).wait()


@jax.jit
def kernel(g, table):
    tblT = jnp.pad(table.T, ((0, 0), (0, _CSTRIDE - table.shape[0]))).reshape(-1)
    mesh = plsc.VectorSubcoreMesh(core_axis_name="c", subcore_axis_name="s")
    run = pl.kernel(
        _body,
        mesh=mesh,
        out_type=jax.ShapeDtypeStruct((_T, 4, _B), jnp.float32),
        scratch_types=[
            pltpu.VMEM((_BQ,), jnp.int32),
            pltpu.VMEM((_BQ,), jnp.int32),
            pltpu.VMEM((4, _BQ), jnp.float32),
            pltpu.VMEM((4, _BQ), jnp.float32),
            pltpu.VMEM((4 * _CSTRIDE,), jnp.float32),
            pltpu.SemaphoreType.DMA,
            pltpu.SemaphoreType.DMA,
            pltpu.SemaphoreType.DMA,
            pltpu.SemaphoreType.DMA,
        ],
        compiler_params=pltpu.CompilerParams(needs_layout_passes=False),
    )
    outP = run(g.T, tblT)
    return outP.transpose(2, 0, 1)
